# Initial kernel scaffold; baseline (speedup 1.0000x reference)
import jax
import jax.numpy as jnp
from jax.experimental import pallas as pl


def kernel(x, table):
    raise NotImplementedError("write your pallas kernel here")


# speedup vs baseline: 2.9643x; 2.9643x over previous
import functools

import jax
import jax.numpy as jnp
from jax import lax
from jax.experimental import pallas as pl
from jax.experimental.pallas import tpu as pltpu
from jax.experimental.pallas import tpu_sc as plsc

B, S = 4096, 50
E = 128
NW = 32
TOTAL = B * S
PER_W = TOTAL // NW
CHUNK = 128
NJ = PER_W // CHUNK


def _make_kernel():
    mesh = plsc.VectorSubcoreMesh(core_axis_name="c", subcore_axis_name="s")

    @functools.partial(
        pl.kernel,
        mesh=mesh,
        out_type=jax.ShapeDtypeStruct((NW, NJ, CHUNK, E), jnp.float32),
        scratch_types=[
            pltpu.VMEM((NJ, CHUNK), jnp.int32),
            pltpu.VMEM((CHUNK, E), jnp.float32),
            pltpu.SemaphoreType.DMA,
        ],
    )
    def k(idx_hbm, table_hbm, out_hbm, idx_v, rows_v, sem):
        wid = lax.axis_index("s") * 2 + lax.axis_index("c")
        pltpu.sync_copy(idx_hbm.at[wid], idx_v)

        def body(j, carry):
            pltpu.async_copy(table_hbm.at[idx_v.at[j]], rows_v, sem).wait()
            pltpu.sync_copy(rows_v, out_hbm.at[wid].at[j])
            return carry

        lax.fori_loop(0, NJ, body, 0)

    return k


_kernel = _make_kernel()


@jax.jit
def kernel(x, table):
    idx = x.reshape(NW, NJ, CHUNK)
    out = _kernel(idx, table)
    return out.reshape(B, S, E)


# speedup vs baseline: 3.3111x; 1.1170x over previous
import functools

import jax
import jax.numpy as jnp
from jax import lax
from jax.experimental import pallas as pl
from jax.experimental.pallas import tpu as pltpu
from jax.experimental.pallas import tpu_sc as plsc

B, S = 4096, 50
E = 128
NW = 32
TOTAL = B * S
PER_W = TOTAL // NW
CHUNK = 128
NJ = PER_W // CHUNK


NBUF = 5
NSTEPS = NJ // NBUF


def _make_kernel():
    mesh = plsc.VectorSubcoreMesh(core_axis_name="c", subcore_axis_name="s")

    @functools.partial(
        pl.kernel,
        mesh=mesh,
        out_type=jax.ShapeDtypeStruct((NW, NJ, CHUNK, E), jnp.float32),
        scratch_types=(
            [pltpu.VMEM((NJ, CHUNK), jnp.int32)]
            + [pltpu.VMEM((CHUNK, E), jnp.float32) for _ in range(NBUF)]
            + [pltpu.SemaphoreType.DMA for _ in range(2 * NBUF)]
        ),
    )
    def k(idx_hbm, table_hbm, out_hbm, idx_v, *rest):
        bufs = rest[:NBUF]
        gsem = rest[NBUF:2 * NBUF]
        osem = rest[2 * NBUF:]
        wid = lax.axis_index("s") * 2 + lax.axis_index("c")
        pltpu.sync_copy(idx_hbm.at[wid], idx_v)

        for b in range(NBUF):
            pltpu.async_copy(table_hbm.at[idx_v.at[b]], bufs[b], gsem[b])

        def body(i, carry):
            j0 = i * NBUF
            for b in range(NBUF):
                j = j0 + b
                pltpu.make_async_copy(
                    table_hbm.at[idx_v.at[j]], bufs[b], gsem[b]).wait()
                pltpu.async_copy(bufs[b], out_hbm.at[wid].at[j], osem[b])
            for b in range(NBUF):
                j = j0 + b
                pltpu.make_async_copy(
                    bufs[b], out_hbm.at[wid].at[j], osem[b]).wait()
                pltpu.async_copy(
                    table_hbm.at[idx_v.at[j + NBUF]], bufs[b], gsem[b])
            return carry

        lax.fori_loop(0, NSTEPS - 1, body, 0)

        j0 = (NSTEPS - 1) * NBUF
        for b in range(NBUF):
            j = j0 + b
            pltpu.make_async_copy(
                table_hbm.at[idx_v.at[j]], bufs[b], gsem[b]).wait()
            pltpu.async_copy(bufs[b], out_hbm.at[wid].at[j], osem[b])
        for b in range(NBUF):
            j = j0 + b
            pltpu.make_async_copy(
                bufs[b], out_hbm.at[wid].at[j], osem[b]).wait()

    return k


_kernel = _make_kernel()


@jax.jit
def kernel(x, table):
    idx = x.reshape(NW, NJ, CHUNK)
    out = _kernel(idx, table)
    return out.reshape(B, S, E)


# speedup vs baseline: 5.9361x; 1.7928x over previous
import functools

import jax
import jax.numpy as jnp
from jax import lax
from jax.experimental import pallas as pl
from jax.experimental.pallas import tpu as pltpu
from jax.experimental.pallas import tpu_sc as plsc

B, S = 4096, 50
E = 128
NW = 32
BPW = B // NW
RBUF = 8
NSTEP = BPW // RBUF


def _make_kernel():
    mesh = plsc.VectorSubcoreMesh(core_axis_name="c", subcore_axis_name="s")

    @functools.partial(
        pl.kernel,
        mesh=mesh,
        out_type=jax.ShapeDtypeStruct((B, S, E), jnp.float32),
        scratch_types=(
            [pltpu.VMEM((BPW, S), jnp.int32)]
            + [pltpu.VMEM((S, E), jnp.float32) for _ in range(RBUF)]
            + [pltpu.SemaphoreType.DMA for _ in range(2 * RBUF)]
        ),
        compiler_params=pltpu.CompilerParams(use_tc_tiling_on_sc=True),
    )
    def k(idx_hbm, table_hbm, out_hbm, idx_v, *rest):
        bufs = rest[:RBUF]
        gsem = rest[RBUF:2 * RBUF]
        osem = rest[2 * RBUF:]
        wid = lax.axis_index("s") * 2 + lax.axis_index("c")
        b0 = wid * BPW
        pltpu.sync_copy(idx_hbm.at[wid], idx_v)

        for r in range(RBUF):
            pltpu.async_copy(table_hbm.at[idx_v.at[r]], bufs[r], gsem[r])

        def body(i, carry):
            j0 = i * RBUF
            for r in range(RBUF):
                j = j0 + r
                pltpu.make_async_copy(
                    table_hbm.at[idx_v.at[j]], bufs[r], gsem[r]).wait()
                pltpu.async_copy(bufs[r], out_hbm.at[b0 + j], osem[r])
            for r in range(RBUF):
                j = j0 + r
                pltpu.make_async_copy(
                    bufs[r], out_hbm.at[b0 + j], osem[r]).wait()
                pltpu.async_copy(
                    table_hbm.at[idx_v.at[j + RBUF]], bufs[r], gsem[r])
            return carry

        lax.fori_loop(0, NSTEP - 1, body, 0)

        j0 = (NSTEP - 1) * RBUF
        for r in range(RBUF):
            j = j0 + r
            pltpu.make_async_copy(
                table_hbm.at[idx_v.at[j]], bufs[r], gsem[r]).wait()
            pltpu.async_copy(bufs[r], out_hbm.at[b0 + j], osem[r])
        for r in range(RBUF):
            j = j0 + r
            pltpu.make_async_copy(
                bufs[r], out_hbm.at[b0 + j], osem[r]).wait()

    return k


_kernel = _make_kernel()


@jax.jit
def kernel(x, table):
    idx = x.reshape(NW, BPW, S)
    return _kernel(idx, table)
